# baseline (speedup 1.0000x reference)
import jax
import jax.numpy as jnp
from jax import lax
from jax.experimental import pallas as pl
from jax.experimental.pallas import tpu as pltpu

_N = 8192
_M = 8192
_BN = 512
_C = 512
_NB = _N // _BN
_NC = _M // _C


def _cand_kernel(p1_ref, aux_ref, u_ref):
    x1 = p1_ref[:, 0:1]
    y1 = p1_ref[:, 1:2]
    dx = x1 - aux_ref[0:1, :]
    dy = y1 - aux_ref[1:2, :]
    t = dx * dx + dy * dy
    m = jnp.min(t, axis=1, keepdims=True)
    u_ref[:, :] = jnp.sqrt(m)


def _band_kernel(bounds_ref, p1_ref, aux_ref, out_ref):
    b = pl.program_id(0)
    jlo = bounds_ref[b, 0]
    jhi = bounds_ref[b, 1]
    x1 = p1_ref[:, 0:1]
    y1 = p1_ref[:, 1:2]

    def body(j, m):
        sl = pl.ds(j * _C, _C)
        dx = x1 - aux_ref[0:1, sl]
        dy = y1 - aux_ref[1:2, sl]
        t = dx * dx + dy * dy
        return jnp.minimum(m, jnp.min(t, axis=1, keepdims=True))

    m = lax.fori_loop(jlo, jhi, body, jnp.full((_BN, 1), jnp.inf, jnp.float32))
    s = jnp.sum(jnp.sqrt(m), keepdims=True).reshape(1, 1) * (1.0 / _N)

    @pl.when(b == 0)
    def _init():
        out_ref[:, :] = jnp.zeros((1, 1), jnp.float32)

    out_ref[:, :] += s


def kernel(pos1, pos2):
    p1s = pos1[jnp.argsort(pos1[:, 0])]
    p2s = pos2[jnp.argsort(pos2[:, 0])]
    x2 = p2s[:, 0]
    y2 = p2s[:, 1]
    aux = jnp.stack([x2, y2])

    u = pl.pallas_call(
        _cand_kernel,
        grid=(_NB,),
        in_specs=[
            pl.BlockSpec((_BN, 2), lambda i: (i, 0)),
            pl.BlockSpec((2, _C), lambda i: (0, i)),
        ],
        out_specs=pl.BlockSpec((_BN, 1), lambda i: (i, 0)),
        out_shape=jax.ShapeDtypeStruct((_N, 1), jnp.float32),
    )(p1s, aux)

    ub = jnp.max(u.reshape(_NB, _BN), axis=1) * 1.000001 + 1e-30
    x1v = p1s[:, 0]
    x1lo = x1v[::_BN]
    x1hi = x1v[_BN - 1::_BN]
    cmin = x2[::_C]
    cmax = x2[_C - 1::_C]
    jlo = jnp.searchsorted(cmax, x1lo - ub, side="left").astype(jnp.int32)
    jhi = jnp.searchsorted(cmin, x1hi + ub, side="right").astype(jnp.int32)
    blk = jnp.arange(_NB, dtype=jnp.int32)
    jlo = jnp.minimum(jlo, blk)
    jhi = jnp.maximum(jhi, blk + 1)
    bounds = jnp.stack([jlo, jhi], axis=1)

    grid_spec = pltpu.PrefetchScalarGridSpec(
        num_scalar_prefetch=1,
        grid=(_NB,),
        in_specs=[
            pl.BlockSpec((_BN, 2), lambda i, b_ref: (i, 0)),
            pl.BlockSpec((2, _M), lambda i, b_ref: (0, 0)),
        ],
        out_specs=pl.BlockSpec((1, 1), lambda i, b_ref: (0, 0)),
    )
    out = pl.pallas_call(
        _band_kernel,
        grid_spec=grid_spec,
        out_shape=jax.ShapeDtypeStruct((1, 1), jnp.float32),
    )(bounds, p1s, aux)
    return out[0, 0]


# speedup vs baseline: 1.3632x; 1.3632x over previous
import jax
import jax.numpy as jnp
from jax import lax
from jax.experimental import pallas as pl
from jax.experimental.pallas import tpu as pltpu

_N = 8192
_M = 8192
_BN = 512
_C = 512
_NB = _N // _BN
_NC = _M // _C


def _cand_kernel(p1_ref, aux_ref, u_ref):
    x1 = p1_ref[:, 0:1]
    y1 = p1_ref[:, 1:2]
    dx = x1 - aux_ref[0:1, :]
    dy = y1 - aux_ref[1:2, :]
    t = dx * dx + dy * dy
    m = jnp.min(t, axis=1, keepdims=True)
    u_ref[:, :] = jnp.sqrt(m)


def _band_kernel(bounds_ref, p1_ref, aux_ref, out_ref):
    b = pl.program_id(0)
    jlo = bounds_ref[b, 0]
    jhi = bounds_ref[b, 1]
    x1 = p1_ref[:, 0:1]
    y1 = p1_ref[:, 1:2]

    def body(j, m):
        sl = pl.ds(j * _C, _C)
        dx = x1 - aux_ref[0:1, sl]
        dy = y1 - aux_ref[1:2, sl]
        t = dx * dx + dy * dy
        return jnp.minimum(m, jnp.min(t, axis=1, keepdims=True))

    m = lax.fori_loop(jlo, jhi, body, jnp.full((_BN, 1), jnp.inf, jnp.float32))
    s = jnp.sum(jnp.sqrt(m), keepdims=True).reshape(1, 1) * (1.0 / _N)

    @pl.when(b == 0)
    def _init():
        out_ref[:, :] = jnp.zeros((1, 1), jnp.float32)

    out_ref[:, :] += s


def kernel(pos1, pos2):
    x1s, y1s = lax.sort((pos1[:, 0], pos1[:, 1]), dimension=0, num_keys=1)
    x2, y2 = lax.sort((pos2[:, 0], pos2[:, 1]), dimension=0, num_keys=1)
    p1s = jnp.stack([x1s, y1s], axis=1)
    aux = jnp.stack([x2, y2])

    u = pl.pallas_call(
        _cand_kernel,
        grid=(_NB,),
        in_specs=[
            pl.BlockSpec((_BN, 2), lambda i: (i, 0)),
            pl.BlockSpec((2, _C), lambda i: (0, i)),
        ],
        out_specs=pl.BlockSpec((_BN, 1), lambda i: (i, 0)),
        out_shape=jax.ShapeDtypeStruct((_N, 1), jnp.float32),
    )(p1s, aux)

    ub = jnp.max(u.reshape(_NB, _BN), axis=1) * 1.000001 + 1e-30
    x1lo = x1s[::_BN]
    x1hi = x1s[_BN - 1::_BN]
    cmin = x2[::_C]
    cmax = x2[_C - 1::_C]
    jlo = jnp.searchsorted(cmax, x1lo - ub, side="left").astype(jnp.int32)
    jhi = jnp.searchsorted(cmin, x1hi + ub, side="right").astype(jnp.int32)
    blk = jnp.arange(_NB, dtype=jnp.int32)
    jlo = jnp.minimum(jlo, blk)
    jhi = jnp.maximum(jhi, blk + 1)
    bounds = jnp.stack([jlo, jhi], axis=1)

    grid_spec = pltpu.PrefetchScalarGridSpec(
        num_scalar_prefetch=1,
        grid=(_NB,),
        in_specs=[
            pl.BlockSpec((_BN, 2), lambda i, b_ref: (i, 0)),
            pl.BlockSpec((2, _M), lambda i, b_ref: (0, 0)),
        ],
        out_specs=pl.BlockSpec((1, 1), lambda i, b_ref: (0, 0)),
    )
    out = pl.pallas_call(
        _band_kernel,
        grid_spec=grid_spec,
        out_shape=jax.ShapeDtypeStruct((1, 1), jnp.float32),
    )(bounds, p1s, aux)
    return out[0, 0]


# speedup vs baseline: 1.4501x; 1.0637x over previous
import jax
import jax.numpy as jnp
from jax import lax
from jax.experimental import pallas as pl
from jax.experimental.pallas import tpu as pltpu

_N = 8192
_M = 8192
_BN = 512
_C = 512
_NB = _N // _BN
_NC = _M // _C


def _cand_kernel(p1_ref, aux_ref, u2_ref):
    x1 = p1_ref[:, 0:1]
    y1 = p1_ref[:, 1:2]
    dx = x1 - aux_ref[0, 0:1, :]
    dy = y1 - aux_ref[0, 1:2, :]
    t = dx * dx + dy * dy
    u2_ref[:, :] = jnp.min(t, axis=1, keepdims=True)


def _band_kernel(bounds_ref, p1_ref, u2_ref, aux_ref, out_ref):
    b = pl.program_id(0)
    jlo = bounds_ref[b, 0]
    jhi = bounds_ref[b, 1]
    x1 = p1_ref[:, 0:1]
    y1 = p1_ref[:, 1:2]

    def body(j, m):
        dx = x1 - aux_ref[j, 0:1, :]
        dy = y1 - aux_ref[j, 1:2, :]
        t = dx * dx + dy * dy
        return jnp.minimum(m, jnp.min(t, axis=1, keepdims=True))

    m = u2_ref[:, :]
    m = lax.fori_loop(jlo, jnp.minimum(jhi, b), body, m)
    m = lax.fori_loop(jnp.maximum(jlo, b + 1), jhi, body, m)
    s = jnp.sum(jnp.sqrt(m), keepdims=True).reshape(1, 1) * (1.0 / _N)

    @pl.when(b == 0)
    def _init():
        out_ref[:, :] = jnp.zeros((1, 1), jnp.float32)

    out_ref[:, :] += s


def kernel(pos1, pos2):
    x1s, y1s = lax.sort((pos1[:, 0], pos1[:, 1]), dimension=0, num_keys=1)
    x2, y2 = lax.sort((pos2[:, 0], pos2[:, 1]), dimension=0, num_keys=1)
    p1s = jnp.stack([x1s, y1s], axis=1)
    aux = jnp.stack([x2.reshape(_NC, _C), y2.reshape(_NC, _C)], axis=1)

    u2 = pl.pallas_call(
        _cand_kernel,
        grid=(_NB,),
        in_specs=[
            pl.BlockSpec((_BN, 2), lambda i: (i, 0)),
            pl.BlockSpec((1, 2, _C), lambda i: (i, 0, 0)),
        ],
        out_specs=pl.BlockSpec((_BN, 1), lambda i: (i, 0)),
        out_shape=jax.ShapeDtypeStruct((_N, 1), jnp.float32),
    )(p1s, aux)

    ub = jnp.sqrt(jnp.max(u2.reshape(_NB, _BN), axis=1)) * 1.000001 + 1e-30
    x1lo = x1s[::_BN]
    x1hi = x1s[_BN - 1::_BN]
    cmin = x2[::_C]
    cmax = x2[_C - 1::_C]
    jlo = jnp.searchsorted(cmax, x1lo - ub, side="left").astype(jnp.int32)
    jhi = jnp.searchsorted(cmin, x1hi + ub, side="right").astype(jnp.int32)
    bounds = jnp.stack([jlo, jhi], axis=1)

    grid_spec = pltpu.PrefetchScalarGridSpec(
        num_scalar_prefetch=1,
        grid=(_NB,),
        in_specs=[
            pl.BlockSpec((_BN, 2), lambda i, b_ref: (i, 0)),
            pl.BlockSpec((_BN, 1), lambda i, b_ref: (i, 0)),
            pl.BlockSpec((_NC, 2, _C), lambda i, b_ref: (0, 0, 0)),
        ],
        out_specs=pl.BlockSpec((1, 1), lambda i, b_ref: (0, 0)),
    )
    out = pl.pallas_call(
        _band_kernel,
        grid_spec=grid_spec,
        out_shape=jax.ShapeDtypeStruct((1, 1), jnp.float32),
    )(bounds, p1s, u2, aux)
    return out[0, 0]


# speedup vs baseline: 2.1431x; 1.4779x over previous
import jax
import jax.numpy as jnp
from jax.experimental import pallas as pl

_N = 8192
_M = 8192
_BN = 2048
_BM = 2048


def _psl_kernel(p1_ref, aux_ref, out_ref):
    x1 = p1_ref[:, 0:1]
    y1 = p1_ref[:, 1:2]
    n1 = x1 * x1 + y1 * y1
    m = None
    for j in range(_M // _BM):
        lo, hi = j * _BM, (j + 1) * _BM
        xs = aux_ref[0:1, lo:hi]
        ys = aux_ref[1:2, lo:hi]
        b2 = aux_ref[2:3, lo:hi]
        t = x1 * xs + (y1 * ys + b2)
        cm = jnp.min(t, axis=1, keepdims=True)
        m = cm if m is None else jnp.minimum(m, cm)
    d2 = jnp.maximum(m + n1, 0.0)
    s = jnp.sum(jnp.sqrt(d2), keepdims=True).reshape(1, 1) * (1.0 / _N)

    @pl.when(pl.program_id(0) == 0)
    def _init():
        out_ref[:, :] = jnp.zeros((1, 1), jnp.float32)

    out_ref[:, :] += s


def kernel(pos1, pos2):
    x2 = pos2[:, 0]
    y2 = pos2[:, 1]
    aux = jnp.stack([-2.0 * x2, -2.0 * y2, x2 * x2 + y2 * y2])
    out = pl.pallas_call(
        _psl_kernel,
        grid=(_N // _BN,),
        in_specs=[
            pl.BlockSpec((_BN, 2), lambda i: (i, 0)),
            pl.BlockSpec((3, _M), lambda i: (0, 0)),
        ],
        out_specs=pl.BlockSpec((1, 1), lambda i: (0, 0)),
        out_shape=jax.ShapeDtypeStruct((1, 1), jnp.float32),
    )(pos1, aux)
    return out[0, 0]


# speedup vs baseline: 2.1638x; 1.0097x over previous
import jax
import jax.numpy as jnp
from jax.experimental import pallas as pl

_N = 8192
_M = 8192
_BN = 2048
_BM = 1024


def _psl_kernel(p1_ref, aux_ref, out_ref):
    x1 = p1_ref[:, 0:1]
    y1 = p1_ref[:, 1:2]
    n1 = x1 * x1 + y1 * y1
    m = None
    for j in range(_M // _BM):
        lo, hi = j * _BM, (j + 1) * _BM
        xs = aux_ref[0:1, lo:hi]
        ys = aux_ref[1:2, lo:hi]
        b2 = aux_ref[2:3, lo:hi]
        t = x1 * xs + (y1 * ys + b2)
        cm = jnp.min(t, axis=1, keepdims=True)
        m = cm if m is None else jnp.minimum(m, cm)
    d2 = jnp.maximum(m + n1, 0.0)
    s = jnp.sum(jnp.sqrt(d2), keepdims=True).reshape(1, 1) * (1.0 / _N)

    @pl.when(pl.program_id(0) == 0)
    def _init():
        out_ref[:, :] = jnp.zeros((1, 1), jnp.float32)

    out_ref[:, :] += s


def kernel(pos1, pos2):
    x2 = pos2[:, 0]
    y2 = pos2[:, 1]
    aux = jnp.stack([-2.0 * x2, -2.0 * y2, x2 * x2 + y2 * y2])
    out = pl.pallas_call(
        _psl_kernel,
        grid=(_N // _BN,),
        in_specs=[
            pl.BlockSpec((_BN, 2), lambda i: (i, 0)),
            pl.BlockSpec((3, _M), lambda i: (0, 0)),
        ],
        out_specs=pl.BlockSpec((1, 1), lambda i: (0, 0)),
        out_shape=jax.ShapeDtypeStruct((1, 1), jnp.float32),
    )(pos1, aux)
    return out[0, 0]
